# baseline (speedup 1.0000x reference)
import functools

import jax
import jax.numpy as jnp
from jax import lax
from jax.experimental import pallas as pl
from jax.experimental.pallas import tpu as pltpu
import jax.experimental.pallas.tpu_sc as plsc

NC = 2
NS = 16
NW = NC * NS
CHUNK = 128
IB = 16


def _sc_mesh():
    return plsc.VectorSubcoreMesh(
        core_axis_name="c", subcore_axis_name="s", num_cores=NC, num_subcores=NS
    )


def _make_deg_kernel(n_pad, cw, rt):
    @functools.partial(
        pl.kernel,
        mesh=_sc_mesh(),
        out_type=jax.ShapeDtypeStruct((NC, 1, n_pad), jnp.float32),
        scratch_types=[
            pltpu.VMEM((cw, CHUNK), jnp.int32),
            pltpu.VMEM((CHUNK,), jnp.float32),
            pltpu.VMEM((rt,), jnp.float32),
            pltpu.VMEM_SHARED((n_pad,), jnp.float32),
        ],
    )
    def deg_k(col_hbm, out_hbm, col_v, ones_v, z_v, accd):
        cid = lax.axis_index("c")
        sid = lax.axis_index("s")
        wid = sid * NC + cid
        pltpu.sync_copy(col_hbm.at[wid], col_v)
        for k in range(CHUNK // 16):
            ones_v[pl.ds(16 * k, 16)] = jnp.full((16,), 1.0, jnp.float32)

        def zb(i, c):
            z_v[pl.ds(i * 16, 16)] = jnp.zeros((16,), jnp.float32)
            return c

        lax.fori_loop(0, rt // 16, zb, 0)
        pltpu.sync_copy(z_v, accd.at[pl.ds(sid * rt, rt)])
        plsc.subcore_barrier()

        def body(j, c):
            pltpu.sync_copy(ones_v, accd.at[col_v.at[j]], add=True)
            return c

        lax.fori_loop(0, cw, body, 0)
        plsc.subcore_barrier()
        pltpu.sync_copy(
            accd.at[pl.ds(sid * rt, rt)], out_hbm.at[cid, 0, pl.ds(sid * rt, rt)]
        )

    return deg_k


def _make_agg_kernel(n_pad, hid, cw, rt):
    assert cw % IB == 0

    @functools.partial(
        pl.kernel,
        mesh=_sc_mesh(),
        out_type=jax.ShapeDtypeStruct((NC, n_pad, hid), jnp.float32),
        scratch_types=[
            pltpu.VMEM((IB, CHUNK), jnp.int32),
            pltpu.VMEM((IB, CHUNK), jnp.int32),
            pltpu.VMEM((2, CHUNK, hid), jnp.float32),
            pltpu.VMEM_SHARED((n_pad, hid), jnp.float32),
            pltpu.SemaphoreType.DMA,
            pltpu.SemaphoreType.DMA,
        ],
    )
    def agg_k(g_hbm, row_hbm, col_hbm, zrows_hbm, out_hbm, row_v, col_v, buf,
              acc, gsem, ssem):
        cid = lax.axis_index("c")
        sid = lax.axis_index("s")
        wid = sid * NC + cid
        pltpu.sync_copy(zrows_hbm, acc.at[pl.ds(sid * rt, rt)])
        plsc.subcore_barrier()

        def blk(b, c):
            pltpu.sync_copy(row_hbm.at[wid, pl.ds(b * IB, IB)], row_v)
            pltpu.sync_copy(col_hbm.at[wid, pl.ds(b * IB, IB)], col_v)
            gd = [None] * IB
            sd = [None] * IB
            gd[0] = pltpu.async_copy(g_hbm.at[row_v.at[0]], buf.at[0], gsem)
            gd[1] = pltpu.async_copy(g_hbm.at[row_v.at[1]], buf.at[1], gsem)
            for j in range(IB):
                bb = j % 2
                gd[j].wait()
                sd[j] = pltpu.async_copy(
                    buf.at[bb], acc.at[col_v.at[j]], ssem, add=True
                )
                if j >= 1:
                    sd[j - 1].wait()
                    if j + 1 < IB:
                        gd[j + 1] = pltpu.async_copy(
                            g_hbm.at[row_v.at[j + 1]], buf.at[(j + 1) % 2], gsem
                        )
            sd[IB - 1].wait()
            return c

        lax.fori_loop(0, cw // IB, blk, 0)
        plsc.subcore_barrier()
        pltpu.sync_copy(
            acc.at[pl.ds(sid * rt, rt)],
            out_hbm.at[cid, pl.ds(sid * rt, rt)],
        )

    return agg_k


def _tc_matmul(x, W1, n_pad, bn):
    f_in = x.shape[1]
    hid = W1.shape[1]

    def body(x_ref, w_ref, h_ref):
        h_ref[...] = jnp.dot(
            x_ref[...], w_ref[...], preferred_element_type=jnp.float32
        )

    return pl.pallas_call(
        body,
        grid=(n_pad // bn,),
        in_specs=[
            pl.BlockSpec((bn, f_in), lambda i: (i, 0)),
            pl.BlockSpec((f_in, hid), lambda i: (0, 0)),
        ],
        out_specs=pl.BlockSpec((bn, hid), lambda i: (i, 0)),
        out_shape=jax.ShapeDtypeStruct((n_pad, hid), jnp.float32),
    )(x, W1)


def _tc_scale(h, degp, n_pad, bn):
    hid = h.shape[1]

    def body(h_ref, dp_ref, g_ref):
        deg = dp_ref[0, :] + dp_ref[1, :] + 1.0
        dis = jnp.where(deg > 0, lax.rsqrt(deg), 0.0)
        g_ref[...] = h_ref[...] * dis[:, None]

    return pl.pallas_call(
        body,
        grid=(n_pad // bn,),
        in_specs=[
            pl.BlockSpec((bn, hid), lambda i: (i, 0)),
            pl.BlockSpec((2, bn), lambda i: (0, i)),
        ],
        out_specs=pl.BlockSpec((bn, hid), lambda i: (i, 0)),
        out_shape=jax.ShapeDtypeStruct((n_pad, hid), jnp.float32),
    )(h, degp)


def _tc_head(aggp, g, degp, b1, W2, b2, n, n_pad, bn):
    hid = g.shape[1]
    ncls = W2.shape[1]

    def body(a_ref, g_ref, dp_ref, b1_ref, w2_ref, b2_ref, o_ref):
        deg = dp_ref[0, :] + dp_ref[1, :] + 1.0
        dis = jnp.where(deg > 0, lax.rsqrt(deg), 0.0)
        s = a_ref[0] + a_ref[1] + g_ref[...]
        h = s * dis[:, None] + b1_ref[0, :][None, :]
        h = jnp.maximum(h, 0.0)
        logits = (
            jnp.dot(h, w2_ref[...], preferred_element_type=jnp.float32)
            + b2_ref[0, :][None, :]
        )
        m = jnp.max(logits, axis=1, keepdims=True)
        ex = logits - m
        lse = jnp.log(jnp.sum(jnp.exp(ex), axis=1, keepdims=True))
        o_ref[...] = ex - lse

    return pl.pallas_call(
        body,
        grid=(n_pad // bn,),
        in_specs=[
            pl.BlockSpec((2, bn, hid), lambda i: (0, i, 0)),
            pl.BlockSpec((bn, hid), lambda i: (i, 0)),
            pl.BlockSpec((2, bn), lambda i: (0, i)),
            pl.BlockSpec((1, hid), lambda i: (0, 0)),
            pl.BlockSpec((hid, ncls), lambda i: (0, 0)),
            pl.BlockSpec((1, ncls), lambda i: (0, 0)),
        ],
        out_specs=pl.BlockSpec((bn, ncls), lambda i: (i, 0)),
        out_shape=jax.ShapeDtypeStruct((n, ncls), jnp.float32),
    )(aggp, g, degp, b1.reshape(1, hid), W2, b2.reshape(1, ncls))


def kernel(x, edge_index, W1, b1, W2, b2):
    n, _ = x.shape
    hid = W1.shape[1]
    e = edge_index.shape[1]

    rt = 640
    n_pad = -(-(n + 1) // rt) * rt
    step = IB * NW * CHUNK
    e_pad = -(-e // step) * step
    cw = e_pad // (NW * CHUNK)
    bn = 512

    row = edge_index[0]
    col = edge_index[1]
    pad = e_pad - e
    if pad:
        ar = jnp.arange(pad, dtype=jnp.int32)
        row = jnp.concatenate([row, ar % n])
        col = jnp.concatenate([col, n + ar % (n_pad - n)])
    row3d = row.reshape(NW, cw, CHUNK)
    col3d = col.reshape(NW, cw, CHUNK)

    degp = _make_deg_kernel(n_pad, cw, rt)(col3d).reshape(NC, n_pad)
    h = _tc_matmul(x, W1, n_pad, bn)
    g = _tc_scale(h, degp, n_pad, bn)
    zrows = jnp.zeros((rt, hid), jnp.float32)
    aggp = _make_agg_kernel(n_pad, hid, cw, rt)(g, row3d, col3d, zrows)
    return _tc_head(aggp, g, degp, b1, W2, b2, n, n_pad, bn)


# speedup vs baseline: 1.0627x; 1.0627x over previous
import functools

import jax
import jax.numpy as jnp
from jax import lax
from jax.experimental import pallas as pl
from jax.experimental.pallas import tpu as pltpu
import jax.experimental.pallas.tpu_sc as plsc

NC = 2
NS = 16
NW = NC * NS
CHUNK = 128
IB = 16


def _sc_mesh():
    return plsc.VectorSubcoreMesh(
        core_axis_name="c", subcore_axis_name="s", num_cores=NC, num_subcores=NS
    )


def _make_deg_kernel(n_pad, cw, rt):
    @functools.partial(
        pl.kernel,
        mesh=_sc_mesh(),
        out_type=jax.ShapeDtypeStruct((NC, 1, n_pad), jnp.float32),
        scratch_types=[
            pltpu.VMEM((cw, CHUNK), jnp.int32),
            pltpu.VMEM((CHUNK,), jnp.float32),
            pltpu.VMEM((rt,), jnp.float32),
            pltpu.VMEM_SHARED((n_pad,), jnp.float32),
        ],
    )
    def deg_k(col_hbm, out_hbm, col_v, ones_v, z_v, accd):
        cid = lax.axis_index("c")
        sid = lax.axis_index("s")
        wid = sid * NC + cid
        pltpu.sync_copy(col_hbm.at[wid], col_v)
        for k in range(CHUNK // 16):
            ones_v[pl.ds(16 * k, 16)] = jnp.full((16,), 1.0, jnp.float32)

        def zb(i, c):
            z_v[pl.ds(i * 16, 16)] = jnp.zeros((16,), jnp.float32)
            return c

        lax.fori_loop(0, rt // 16, zb, 0)
        pltpu.sync_copy(z_v, accd.at[pl.ds(sid * rt, rt)])
        plsc.subcore_barrier()

        def body(j, c):
            pltpu.sync_copy(ones_v, accd.at[col_v.at[j]], add=True)
            return c

        lax.fori_loop(0, cw, body, 0)
        plsc.subcore_barrier()
        pltpu.sync_copy(
            accd.at[pl.ds(sid * rt, rt)], out_hbm.at[cid, 0, pl.ds(sid * rt, rt)]
        )

    return deg_k


def _make_agg_kernel(n_pad, hid, cw, rt):
    assert cw % IB == 0

    @functools.partial(
        pl.kernel,
        mesh=_sc_mesh(),
        out_type=jax.ShapeDtypeStruct((NC, n_pad, hid), jnp.float32),
        scratch_types=[
            pltpu.VMEM((IB, CHUNK), jnp.int32),
            pltpu.VMEM((IB, CHUNK), jnp.int32),
            pltpu.VMEM((2, CHUNK, hid), jnp.float32),
            pltpu.VMEM_SHARED((n_pad, hid), jnp.float32),
            pltpu.SemaphoreType.DMA,
            pltpu.SemaphoreType.DMA,
        ],
    )
    def agg_k(g_hbm, row_hbm, col_hbm, zrows_hbm, out_hbm, row_v, col_v, buf,
              acc, gsem, ssem):
        cid = lax.axis_index("c")
        sid = lax.axis_index("s")
        wid = sid * NC + cid
        pltpu.sync_copy(zrows_hbm, acc.at[pl.ds(sid * rt, rt)])
        plsc.subcore_barrier()

        def blk(b, c):
            pltpu.sync_copy(row_hbm.at[wid, pl.ds(b * IB, IB)], row_v)
            pltpu.sync_copy(col_hbm.at[wid, pl.ds(b * IB, IB)], col_v)
            pltpu.async_copy(g_hbm.at[row_v.at[0]], buf.at[0], gsem)

            def body(k, c2):
                j0 = 2 * k
                pltpu.make_async_copy(
                    g_hbm.at[row_v.at[j0]], buf.at[0], gsem
                ).wait()
                pltpu.async_copy(g_hbm.at[row_v.at[j0 + 1]], buf.at[1], gsem)
                pltpu.sync_copy(buf.at[0], acc.at[col_v.at[j0]], add=True)

                @pl.when(j0 + 2 < IB)
                def _():
                    pltpu.async_copy(g_hbm.at[row_v.at[j0 + 2]], buf.at[0], gsem)

                pltpu.make_async_copy(
                    g_hbm.at[row_v.at[j0 + 1]], buf.at[1], gsem
                ).wait()
                pltpu.sync_copy(buf.at[1], acc.at[col_v.at[j0 + 1]], add=True)
                return c2

            lax.fori_loop(0, IB // 2, body, 0)
            return c

        lax.fori_loop(0, cw // IB, blk, 0)
        plsc.subcore_barrier()
        pltpu.sync_copy(
            acc.at[pl.ds(sid * rt, rt)],
            out_hbm.at[cid, pl.ds(sid * rt, rt)],
        )

    return agg_k


def _tc_matmul(x, W1, n_pad, bn):
    f_in = x.shape[1]
    hid = W1.shape[1]

    def body(x_ref, w_ref, h_ref):
        h_ref[...] = jnp.dot(
            x_ref[...], w_ref[...], preferred_element_type=jnp.float32
        )

    return pl.pallas_call(
        body,
        grid=(n_pad // bn,),
        in_specs=[
            pl.BlockSpec((bn, f_in), lambda i: (i, 0)),
            pl.BlockSpec((f_in, hid), lambda i: (0, 0)),
        ],
        out_specs=pl.BlockSpec((bn, hid), lambda i: (i, 0)),
        out_shape=jax.ShapeDtypeStruct((n_pad, hid), jnp.float32),
    )(x, W1)


def _tc_scale(h, degp, n_pad, bn):
    hid = h.shape[1]

    def body(h_ref, dp_ref, g_ref):
        deg = dp_ref[0, :] + dp_ref[1, :] + 1.0
        dis = jnp.where(deg > 0, lax.rsqrt(deg), 0.0)
        g_ref[...] = h_ref[...] * dis[:, None]

    return pl.pallas_call(
        body,
        grid=(n_pad // bn,),
        in_specs=[
            pl.BlockSpec((bn, hid), lambda i: (i, 0)),
            pl.BlockSpec((2, bn), lambda i: (0, i)),
        ],
        out_specs=pl.BlockSpec((bn, hid), lambda i: (i, 0)),
        out_shape=jax.ShapeDtypeStruct((n_pad, hid), jnp.float32),
    )(h, degp)


def _tc_head(aggp, g, degp, b1, W2, b2, n, n_pad, bn):
    hid = g.shape[1]
    ncls = W2.shape[1]

    def body(a_ref, g_ref, dp_ref, b1_ref, w2_ref, b2_ref, o_ref):
        deg = dp_ref[0, :] + dp_ref[1, :] + 1.0
        dis = jnp.where(deg > 0, lax.rsqrt(deg), 0.0)
        s = a_ref[0] + a_ref[1] + g_ref[...]
        h = s * dis[:, None] + b1_ref[0, :][None, :]
        h = jnp.maximum(h, 0.0)
        logits = (
            jnp.dot(h, w2_ref[...], preferred_element_type=jnp.float32)
            + b2_ref[0, :][None, :]
        )
        m = jnp.max(logits, axis=1, keepdims=True)
        ex = logits - m
        lse = jnp.log(jnp.sum(jnp.exp(ex), axis=1, keepdims=True))
        o_ref[...] = ex - lse

    return pl.pallas_call(
        body,
        grid=(n_pad // bn,),
        in_specs=[
            pl.BlockSpec((2, bn, hid), lambda i: (0, i, 0)),
            pl.BlockSpec((bn, hid), lambda i: (i, 0)),
            pl.BlockSpec((2, bn), lambda i: (0, i)),
            pl.BlockSpec((1, hid), lambda i: (0, 0)),
            pl.BlockSpec((hid, ncls), lambda i: (0, 0)),
            pl.BlockSpec((1, ncls), lambda i: (0, 0)),
        ],
        out_specs=pl.BlockSpec((bn, ncls), lambda i: (i, 0)),
        out_shape=jax.ShapeDtypeStruct((n, ncls), jnp.float32),
    )(aggp, g, degp, b1.reshape(1, hid), W2, b2.reshape(1, ncls))


def kernel(x, edge_index, W1, b1, W2, b2):
    n, _ = x.shape
    hid = W1.shape[1]
    e = edge_index.shape[1]

    rt = 640
    n_pad = -(-(n + 1) // rt) * rt
    step = IB * NW * CHUNK
    e_pad = -(-e // step) * step
    cw = e_pad // (NW * CHUNK)
    bn = 512

    row = edge_index[0]
    col = edge_index[1]
    pad = e_pad - e
    if pad:
        ar = jnp.arange(pad, dtype=jnp.int32)
        row = jnp.concatenate([row, ar % n])
        col = jnp.concatenate([col, n + ar % (n_pad - n)])
    row3d = row.reshape(NW, cw, CHUNK)
    col3d = col.reshape(NW, cw, CHUNK)

    degp = _make_deg_kernel(n_pad, cw, rt)(col3d).reshape(NC, n_pad)
    h = _tc_matmul(x, W1, n_pad, bn)
    g = _tc_scale(h, degp, n_pad, bn)
    zrows = jnp.zeros((rt, hid), jnp.float32)
    aggp = _make_agg_kernel(n_pad, hid, cw, rt)(g, row3d, col3d, zrows)
    return _tc_head(aggp, g, degp, b1, W2, b2, n, n_pad, bn)


# speedup vs baseline: 1.0909x; 1.0265x over previous
import functools

import jax
import jax.numpy as jnp
from jax import lax
from jax.experimental import pallas as pl
from jax.experimental.pallas import tpu as pltpu
import jax.experimental.pallas.tpu_sc as plsc

NC = 2
NS = 16
NW = NC * NS
CHUNK = 128
IB = 16


def _sc_mesh():
    return plsc.VectorSubcoreMesh(
        core_axis_name="c", subcore_axis_name="s", num_cores=NC, num_subcores=NS
    )


def _make_deg_kernel(n_pad, cw, rt):
    @functools.partial(
        pl.kernel,
        mesh=_sc_mesh(),
        out_type=jax.ShapeDtypeStruct((NC, 1, n_pad), jnp.float32),
        scratch_types=[
            pltpu.VMEM((cw, CHUNK), jnp.int32),
            pltpu.VMEM((CHUNK,), jnp.float32),
            pltpu.VMEM((rt,), jnp.float32),
            pltpu.VMEM_SHARED((n_pad,), jnp.float32),
        ],
    )
    def deg_k(col_hbm, out_hbm, col_v, ones_v, z_v, accd):
        cid = lax.axis_index("c")
        sid = lax.axis_index("s")
        wid = sid * NC + cid
        pltpu.sync_copy(col_hbm.at[wid], col_v)
        for k in range(CHUNK // 16):
            ones_v[pl.ds(16 * k, 16)] = jnp.full((16,), 1.0, jnp.float32)

        def zb(i, c):
            z_v[pl.ds(i * 16, 16)] = jnp.zeros((16,), jnp.float32)
            return c

        lax.fori_loop(0, rt // 16, zb, 0)
        pltpu.sync_copy(z_v, accd.at[pl.ds(sid * rt, rt)])
        plsc.subcore_barrier()

        def body(j, c):
            pltpu.sync_copy(ones_v, accd.at[col_v.at[j]], add=True)
            return c

        lax.fori_loop(0, cw, body, 0)
        plsc.subcore_barrier()
        pltpu.sync_copy(
            accd.at[pl.ds(sid * rt, rt)], out_hbm.at[cid, 0, pl.ds(sid * rt, rt)]
        )

    return deg_k


def _make_agg_kernel(n_pad, hid, cw, rt):
    assert cw % IB == 0

    @functools.partial(
        pl.kernel,
        mesh=_sc_mesh(),
        out_type=jax.ShapeDtypeStruct((NC, n_pad, hid), jnp.float32),
        scratch_types=[
            pltpu.VMEM((IB, CHUNK), jnp.int32),
            pltpu.VMEM((IB, CHUNK), jnp.int32),
            pltpu.VMEM((2, CHUNK, hid), jnp.float32),
            pltpu.VMEM_SHARED((n_pad, hid), jnp.float32),
            pltpu.SemaphoreType.DMA,
            pltpu.SemaphoreType.DMA,
        ],
    )
    def agg_k(g_hbm, row_hbm, col_hbm, zrows_hbm, out_hbm, row_v, col_v, buf,
              acc, gsem, ssem):
        cid = lax.axis_index("c")
        sid = lax.axis_index("s")
        wid = sid * NC + cid
        pltpu.sync_copy(zrows_hbm, acc.at[pl.ds(sid * rt, rt)])
        pltpu.sync_copy(row_hbm.at[wid, pl.ds(0, IB)], row_v)
        pltpu.sync_copy(col_hbm.at[wid, pl.ds(0, IB)], col_v)
        pltpu.async_copy(g_hbm.at[row_v.at[0]], buf.at[0], gsem)
        plsc.subcore_barrier()

        def blk(b, c):
            @pl.when(b > 0)
            def _():
                pltpu.sync_copy(row_hbm.at[wid, pl.ds(b * IB, IB)], row_v)
                pltpu.sync_copy(col_hbm.at[wid, pl.ds(b * IB, IB)], col_v)
                pltpu.async_copy(g_hbm.at[row_v.at[0]], buf.at[0], gsem)

            def body(k, c2):
                j0 = 2 * k
                pltpu.make_async_copy(
                    g_hbm.at[row_v.at[j0]], buf.at[0], gsem
                ).wait()
                pltpu.async_copy(g_hbm.at[row_v.at[j0 + 1]], buf.at[1], gsem)
                pltpu.sync_copy(buf.at[0], acc.at[col_v.at[j0]], add=True)

                @pl.when(j0 + 2 < IB)
                def _():
                    pltpu.async_copy(g_hbm.at[row_v.at[j0 + 2]], buf.at[0], gsem)

                pltpu.make_async_copy(
                    g_hbm.at[row_v.at[j0 + 1]], buf.at[1], gsem
                ).wait()
                pltpu.sync_copy(buf.at[1], acc.at[col_v.at[j0 + 1]], add=True)
                return c2

            lax.fori_loop(0, IB // 2, body, 0)
            return c

        lax.fori_loop(0, cw // IB, blk, 0)
        plsc.subcore_barrier()
        pltpu.sync_copy(
            acc.at[pl.ds(sid * rt, rt)],
            out_hbm.at[cid, pl.ds(sid * rt, rt)],
        )

    return agg_k


def _tc_scale(x, W1, degp, n_pad, bn):
    f_in = x.shape[1]
    hid = W1.shape[1]

    def body(x_ref, w_ref, dp_ref, g_ref):
        h = jnp.dot(x_ref[...], w_ref[...], preferred_element_type=jnp.float32)
        deg = dp_ref[0, :] + dp_ref[1, :] + 1.0
        dis = jnp.where(deg > 0, lax.rsqrt(deg), 0.0)
        g_ref[...] = h * dis[:, None]

    return pl.pallas_call(
        body,
        grid=(n_pad // bn,),
        in_specs=[
            pl.BlockSpec((bn, f_in), lambda i: (i, 0)),
            pl.BlockSpec((f_in, hid), lambda i: (0, 0)),
            pl.BlockSpec((2, bn), lambda i: (0, i)),
        ],
        out_specs=pl.BlockSpec((bn, hid), lambda i: (i, 0)),
        out_shape=jax.ShapeDtypeStruct((n_pad, hid), jnp.float32),
    )(x, W1, degp)


def _tc_head(aggp, g, degp, b1, W2, b2, n, n_pad, bn):
    hid = g.shape[1]
    ncls = W2.shape[1]

    def body(a_ref, g_ref, dp_ref, b1_ref, w2_ref, b2_ref, o_ref):
        deg = dp_ref[0, :] + dp_ref[1, :] + 1.0
        dis = jnp.where(deg > 0, lax.rsqrt(deg), 0.0)
        s = a_ref[0] + a_ref[1] + g_ref[...]
        h = s * dis[:, None] + b1_ref[0, :][None, :]
        h = jnp.maximum(h, 0.0)
        logits = (
            jnp.dot(h, w2_ref[...], preferred_element_type=jnp.float32)
            + b2_ref[0, :][None, :]
        )
        m = jnp.max(logits, axis=1, keepdims=True)
        ex = logits - m
        lse = jnp.log(jnp.sum(jnp.exp(ex), axis=1, keepdims=True))
        o_ref[...] = ex - lse

    return pl.pallas_call(
        body,
        grid=(n_pad // bn,),
        in_specs=[
            pl.BlockSpec((2, bn, hid), lambda i: (0, i, 0)),
            pl.BlockSpec((bn, hid), lambda i: (i, 0)),
            pl.BlockSpec((2, bn), lambda i: (0, i)),
            pl.BlockSpec((1, hid), lambda i: (0, 0)),
            pl.BlockSpec((hid, ncls), lambda i: (0, 0)),
            pl.BlockSpec((1, ncls), lambda i: (0, 0)),
        ],
        out_specs=pl.BlockSpec((bn, ncls), lambda i: (i, 0)),
        out_shape=jax.ShapeDtypeStruct((n, ncls), jnp.float32),
    )(aggp, g, degp, b1.reshape(1, hid), W2, b2.reshape(1, ncls))


def kernel(x, edge_index, W1, b1, W2, b2):
    n, _ = x.shape
    hid = W1.shape[1]
    e = edge_index.shape[1]

    rt = 640
    n_pad = -(-(n + 1) // rt) * rt
    step = IB * NW * CHUNK
    e_pad = -(-e // step) * step
    cw = e_pad // (NW * CHUNK)
    bn = 512

    row = edge_index[0]
    col = edge_index[1]
    pad = e_pad - e
    if pad:
        ar = jnp.arange(pad, dtype=jnp.int32)
        row = jnp.concatenate([row, ar % n])
        col = jnp.concatenate([col, n + ar % (n_pad - n)])
    row3d = row.reshape(NW, cw, CHUNK)
    col3d = col.reshape(NW, cw, CHUNK)

    degp = _make_deg_kernel(n_pad, cw, rt)(col3d).reshape(NC, n_pad)
    g = _tc_scale(x, W1, degp, n_pad, bn)
    zrows = jnp.zeros((rt, hid), jnp.float32)
    aggp = _make_agg_kernel(n_pad, hid, cw, rt)(g, row3d, col3d, zrows)
    return _tc_head(aggp, g, degp, b1, W2, b2, n, n_pad, bn)


# speedup vs baseline: 1.1188x; 1.0255x over previous
import functools

import jax
import jax.numpy as jnp
from jax import lax
from jax.experimental import pallas as pl
from jax.experimental.pallas import tpu as pltpu
import jax.experimental.pallas.tpu_sc as plsc

NC = 2
NS = 16
NW = NC * NS
CHUNK = 128
IB = 16


def _sc_mesh():
    return plsc.VectorSubcoreMesh(
        core_axis_name="c", subcore_axis_name="s", num_cores=NC, num_subcores=NS
    )


def _make_deg_kernel(n_pad, cw, rt):
    @functools.partial(
        pl.kernel,
        mesh=_sc_mesh(),
        out_type=jax.ShapeDtypeStruct((NC, 1, n_pad), jnp.float32),
        scratch_types=[
            pltpu.VMEM((cw, CHUNK), jnp.int32),
            pltpu.VMEM((CHUNK,), jnp.float32),
            pltpu.VMEM((rt,), jnp.float32),
            pltpu.VMEM_SHARED((n_pad,), jnp.float32),
            pltpu.SemaphoreType.DMA,
        ],
    )
    def deg_k(col_hbm, out_hbm, col_v, ones_v, z_v, accd, ssem):
        cid = lax.axis_index("c")
        sid = lax.axis_index("s")
        wid = sid * NC + cid
        pltpu.sync_copy(col_hbm.at[wid], col_v)
        for k in range(CHUNK // 16):
            ones_v[pl.ds(16 * k, 16)] = jnp.full((16,), 1.0, jnp.float32)

        def zb(i, c):
            z_v[pl.ds(i * 16, 16)] = jnp.zeros((16,), jnp.float32)
            return c

        lax.fori_loop(0, rt // 16, zb, 0)
        pltpu.sync_copy(z_v, accd.at[pl.ds(sid * rt, rt)])
        plsc.subcore_barrier()

        def body(j, c):
            pltpu.async_copy(ones_v, accd.at[col_v.at[j]], ssem, add=True)
            return c

        lax.fori_loop(0, cw, body, 0)

        def drain(j, c):
            pltpu.make_async_copy(ones_v, accd.at[col_v.at[j]], ssem).wait()
            return c

        lax.fori_loop(0, cw, drain, 0)
        plsc.subcore_barrier()
        pltpu.sync_copy(
            accd.at[pl.ds(sid * rt, rt)], out_hbm.at[cid, 0, pl.ds(sid * rt, rt)]
        )

    return deg_k


def _make_agg_kernel(n_pad, hid, cw, rt):
    assert cw % IB == 0

    @functools.partial(
        pl.kernel,
        mesh=_sc_mesh(),
        out_type=jax.ShapeDtypeStruct((NC, n_pad, hid), jnp.float32),
        scratch_types=[
            pltpu.VMEM((IB, CHUNK), jnp.int32),
            pltpu.VMEM((IB, CHUNK), jnp.int32),
            pltpu.VMEM((2, CHUNK, hid), jnp.float32),
            pltpu.VMEM_SHARED((n_pad, hid), jnp.float32),
            pltpu.SemaphoreType.DMA,
            pltpu.SemaphoreType.DMA,
        ],
    )
    def agg_k(g_hbm, row_hbm, col_hbm, zrows_hbm, out_hbm, row_v, col_v, buf,
              acc, gsem, ssem):
        cid = lax.axis_index("c")
        sid = lax.axis_index("s")
        wid = sid * NC + cid
        pltpu.sync_copy(zrows_hbm, acc.at[pl.ds(sid * rt, rt)])
        pltpu.sync_copy(row_hbm.at[wid, pl.ds(0, IB)], row_v)
        pltpu.sync_copy(col_hbm.at[wid, pl.ds(0, IB)], col_v)
        pltpu.async_copy(g_hbm.at[row_v.at[0]], buf.at[0], gsem)
        plsc.subcore_barrier()

        def blk(b, c):
            @pl.when(b > 0)
            def _():
                pltpu.sync_copy(row_hbm.at[wid, pl.ds(b * IB, IB)], row_v)
                pltpu.sync_copy(col_hbm.at[wid, pl.ds(b * IB, IB)], col_v)
                pltpu.async_copy(g_hbm.at[row_v.at[0]], buf.at[0], gsem)

            def body(k, c2):
                j0 = 2 * k
                pltpu.make_async_copy(
                    g_hbm.at[row_v.at[j0]], buf.at[0], gsem
                ).wait()
                pltpu.async_copy(g_hbm.at[row_v.at[j0 + 1]], buf.at[1], gsem)
                pltpu.sync_copy(buf.at[0], acc.at[col_v.at[j0]], add=True)

                @pl.when(j0 + 2 < IB)
                def _():
                    pltpu.async_copy(g_hbm.at[row_v.at[j0 + 2]], buf.at[0], gsem)

                pltpu.make_async_copy(
                    g_hbm.at[row_v.at[j0 + 1]], buf.at[1], gsem
                ).wait()
                pltpu.sync_copy(buf.at[1], acc.at[col_v.at[j0 + 1]], add=True)
                return c2

            lax.fori_loop(0, IB // 2, body, 0)
            return c

        lax.fori_loop(0, cw // IB, blk, 0)
        plsc.subcore_barrier()
        pltpu.sync_copy(
            acc.at[pl.ds(sid * rt, rt)],
            out_hbm.at[cid, pl.ds(sid * rt, rt)],
        )

    return agg_k


def _tc_scale(x, W1, degp, n_pad, bn):
    f_in = x.shape[1]
    hid = W1.shape[1]

    def body(x_ref, w_ref, dp_ref, g_ref):
        h = jnp.dot(x_ref[...], w_ref[...], preferred_element_type=jnp.float32)
        deg = dp_ref[0, :] + dp_ref[1, :] + 1.0
        dis = jnp.where(deg > 0, lax.rsqrt(deg), 0.0)
        g_ref[...] = h * dis[:, None]

    return pl.pallas_call(
        body,
        grid=(n_pad // bn,),
        in_specs=[
            pl.BlockSpec((bn, f_in), lambda i: (i, 0)),
            pl.BlockSpec((f_in, hid), lambda i: (0, 0)),
            pl.BlockSpec((2, bn), lambda i: (0, i)),
        ],
        out_specs=pl.BlockSpec((bn, hid), lambda i: (i, 0)),
        out_shape=jax.ShapeDtypeStruct((n_pad, hid), jnp.float32),
    )(x, W1, degp)


def _tc_head(aggp, g, degp, b1, W2, b2, n, n_pad, bn):
    hid = g.shape[1]
    ncls = W2.shape[1]

    def body(a_ref, g_ref, dp_ref, b1_ref, w2_ref, b2_ref, o_ref):
        deg = dp_ref[0, :] + dp_ref[1, :] + 1.0
        dis = jnp.where(deg > 0, lax.rsqrt(deg), 0.0)
        s = a_ref[0] + a_ref[1] + g_ref[...]
        h = s * dis[:, None] + b1_ref[0, :][None, :]
        h = jnp.maximum(h, 0.0)
        logits = (
            jnp.dot(h, w2_ref[...], preferred_element_type=jnp.float32)
            + b2_ref[0, :][None, :]
        )
        m = jnp.max(logits, axis=1, keepdims=True)
        ex = logits - m
        lse = jnp.log(jnp.sum(jnp.exp(ex), axis=1, keepdims=True))
        o_ref[...] = ex - lse

    return pl.pallas_call(
        body,
        grid=(n_pad // bn,),
        in_specs=[
            pl.BlockSpec((2, bn, hid), lambda i: (0, i, 0)),
            pl.BlockSpec((bn, hid), lambda i: (i, 0)),
            pl.BlockSpec((2, bn), lambda i: (0, i)),
            pl.BlockSpec((1, hid), lambda i: (0, 0)),
            pl.BlockSpec((hid, ncls), lambda i: (0, 0)),
            pl.BlockSpec((1, ncls), lambda i: (0, 0)),
        ],
        out_specs=pl.BlockSpec((bn, ncls), lambda i: (i, 0)),
        out_shape=jax.ShapeDtypeStruct((n, ncls), jnp.float32),
    )(aggp, g, degp, b1.reshape(1, hid), W2, b2.reshape(1, ncls))


def kernel(x, edge_index, W1, b1, W2, b2):
    n, _ = x.shape
    hid = W1.shape[1]
    e = edge_index.shape[1]

    rt = 640
    n_pad = -(-(n + 1) // rt) * rt
    step = IB * NW * CHUNK
    e_pad = -(-e // step) * step
    cw = e_pad // (NW * CHUNK)
    bn = 512

    row = edge_index[0]
    col = edge_index[1]
    pad = e_pad - e
    if pad:
        ar = jnp.arange(pad, dtype=jnp.int32)
        row = jnp.concatenate([row, ar % n])
        col = jnp.concatenate([col, n + ar % (n_pad - n)])
    row3d = row.reshape(NW, cw, CHUNK)
    col3d = col.reshape(NW, cw, CHUNK)

    degp = _make_deg_kernel(n_pad, cw, rt)(col3d).reshape(NC, n_pad)
    g = _tc_scale(x, W1, degp, n_pad, bn)
    zrows = jnp.zeros((rt, hid), jnp.float32)
    aggp = _make_agg_kernel(n_pad, hid, cw, rt)(g, row3d, col3d, zrows)
    return _tc_head(aggp, g, degp, b1, W2, b2, n, n_pad, bn)


# speedup vs baseline: 1.1209x; 1.0019x over previous
import functools

import jax
import jax.numpy as jnp
from jax import lax
from jax.experimental import pallas as pl
from jax.experimental.pallas import tpu as pltpu
import jax.experimental.pallas.tpu_sc as plsc

NC = 2
NS = 16
NW = NC * NS
CHUNK = 128
IB = 16


def _sc_mesh():
    return plsc.VectorSubcoreMesh(
        core_axis_name="c", subcore_axis_name="s", num_cores=NC, num_subcores=NS
    )


def _make_deg_kernel(n_pad, cw, rt):
    @functools.partial(
        pl.kernel,
        mesh=_sc_mesh(),
        out_type=jax.ShapeDtypeStruct((NC, 1, n_pad), jnp.float32),
        scratch_types=[
            pltpu.VMEM((cw, CHUNK), jnp.int32),
            pltpu.VMEM((CHUNK,), jnp.float32),
            pltpu.VMEM((rt,), jnp.float32),
            pltpu.VMEM_SHARED((n_pad,), jnp.float32),
            pltpu.SemaphoreType.DMA,
        ],
    )
    def deg_k(col_hbm, out_hbm, col_v, ones_v, z_v, accd, ssem):
        cid = lax.axis_index("c")
        sid = lax.axis_index("s")
        wid = sid * NC + cid
        pltpu.sync_copy(col_hbm.at[wid], col_v)
        for k in range(CHUNK // 16):
            ones_v[pl.ds(16 * k, 16)] = jnp.full((16,), 1.0, jnp.float32)

        def zb(i, c):
            z_v[pl.ds(i * 16, 16)] = jnp.zeros((16,), jnp.float32)
            return c

        lax.fori_loop(0, rt // 16, zb, 0)
        pltpu.sync_copy(z_v, accd.at[pl.ds(sid * rt, rt)])
        plsc.subcore_barrier()

        def body(j, c):
            pltpu.async_copy(ones_v, accd.at[col_v.at[j]], ssem, add=True)
            return c

        lax.fori_loop(0, cw, body, 0)

        def drain(j, c):
            pltpu.make_async_copy(ones_v, accd.at[col_v.at[j]], ssem).wait()
            return c

        lax.fori_loop(0, cw, drain, 0)
        plsc.subcore_barrier()
        pltpu.sync_copy(
            accd.at[pl.ds(sid * rt, rt)], out_hbm.at[cid, 0, pl.ds(sid * rt, rt)]
        )

    return deg_k


def _make_agg_kernel(n_pad, hid, cw, rt):
    assert cw % IB == 0

    @functools.partial(
        pl.kernel,
        mesh=_sc_mesh(),
        out_type=jax.ShapeDtypeStruct((NC, n_pad, hid), jnp.float32),
        scratch_types=[
            pltpu.VMEM((IB, CHUNK), jnp.int32),
            pltpu.VMEM((IB, CHUNK), jnp.int32),
            pltpu.VMEM((2, CHUNK, hid), jnp.float32),
            pltpu.VMEM_SHARED((n_pad, hid), jnp.float32),
            pltpu.SemaphoreType.DMA,
            pltpu.SemaphoreType.DMA,
        ],
    )
    def agg_k(g_hbm, row_hbm, col_hbm, zrows_hbm, out_hbm, row_v, col_v, buf,
              acc, gsem, ssem):
        cid = lax.axis_index("c")
        sid = lax.axis_index("s")
        wid = sid * NC + cid
        pltpu.sync_copy(zrows_hbm, acc.at[pl.ds(sid * rt, rt)])
        pltpu.sync_copy(row_hbm.at[wid, pl.ds(0, IB)], row_v)
        pltpu.sync_copy(col_hbm.at[wid, pl.ds(0, IB)], col_v)
        pltpu.async_copy(g_hbm.at[row_v.at[0]], buf.at[0], gsem)
        plsc.subcore_barrier()

        def blk(b, c):
            @pl.when(b > 0)
            def _():
                pltpu.sync_copy(row_hbm.at[wid, pl.ds(b * IB, IB)], row_v)
                pltpu.sync_copy(col_hbm.at[wid, pl.ds(b * IB, IB)], col_v)
                pltpu.async_copy(g_hbm.at[row_v.at[0]], buf.at[0], gsem)

            pltpu.async_copy(g_hbm.at[row_v.at[1]], buf.at[1], gsem)

            def body(k, c2):
                j0 = 2 * k
                pltpu.make_async_copy(
                    g_hbm.at[row_v.at[j0]], buf.at[0], gsem
                ).wait()
                pltpu.async_copy(buf.at[0], acc.at[col_v.at[j0]], ssem, add=True)
                pltpu.make_async_copy(
                    g_hbm.at[row_v.at[j0 + 1]], buf.at[1], gsem
                ).wait()
                pltpu.async_copy(buf.at[1], acc.at[col_v.at[j0 + 1]], ssem,
                                 add=True)
                pltpu.make_async_copy(
                    buf.at[0], acc.at[col_v.at[j0]], ssem
                ).wait()

                @pl.when(j0 + 2 < IB)
                def _():
                    pltpu.async_copy(g_hbm.at[row_v.at[j0 + 2]], buf.at[0], gsem)

                pltpu.make_async_copy(
                    buf.at[1], acc.at[col_v.at[j0 + 1]], ssem
                ).wait()

                @pl.when(j0 + 3 < IB)
                def _():
                    pltpu.async_copy(g_hbm.at[row_v.at[j0 + 3]], buf.at[1], gsem)

                return c2

            lax.fori_loop(0, IB // 2, body, 0)
            return c

        lax.fori_loop(0, cw // IB, blk, 0)
        plsc.subcore_barrier()
        pltpu.sync_copy(
            acc.at[pl.ds(sid * rt, rt)],
            out_hbm.at[cid, pl.ds(sid * rt, rt)],
        )

    return agg_k


def _tc_scale(x, W1, degp, n_pad, bn):
    f_in = x.shape[1]
    hid = W1.shape[1]

    def body(x_ref, w_ref, dp_ref, g_ref):
        h = jnp.dot(x_ref[...], w_ref[...], preferred_element_type=jnp.float32)
        deg = dp_ref[0, :] + dp_ref[1, :] + 1.0
        dis = jnp.where(deg > 0, lax.rsqrt(deg), 0.0)
        g_ref[...] = h * dis[:, None]

    return pl.pallas_call(
        body,
        grid=(n_pad // bn,),
        in_specs=[
            pl.BlockSpec((bn, f_in), lambda i: (i, 0)),
            pl.BlockSpec((f_in, hid), lambda i: (0, 0)),
            pl.BlockSpec((2, bn), lambda i: (0, i)),
        ],
        out_specs=pl.BlockSpec((bn, hid), lambda i: (i, 0)),
        out_shape=jax.ShapeDtypeStruct((n_pad, hid), jnp.float32),
    )(x, W1, degp)


def _tc_head(aggp, g, degp, b1, W2, b2, n, n_pad, bn):
    hid = g.shape[1]
    ncls = W2.shape[1]

    def body(a_ref, g_ref, dp_ref, b1_ref, w2_ref, b2_ref, o_ref):
        deg = dp_ref[0, :] + dp_ref[1, :] + 1.0
        dis = jnp.where(deg > 0, lax.rsqrt(deg), 0.0)
        s = a_ref[0] + a_ref[1] + g_ref[...]
        h = s * dis[:, None] + b1_ref[0, :][None, :]
        h = jnp.maximum(h, 0.0)
        logits = (
            jnp.dot(h, w2_ref[...], preferred_element_type=jnp.float32)
            + b2_ref[0, :][None, :]
        )
        m = jnp.max(logits, axis=1, keepdims=True)
        ex = logits - m
        lse = jnp.log(jnp.sum(jnp.exp(ex), axis=1, keepdims=True))
        o_ref[...] = ex - lse

    return pl.pallas_call(
        body,
        grid=(n_pad // bn,),
        in_specs=[
            pl.BlockSpec((2, bn, hid), lambda i: (0, i, 0)),
            pl.BlockSpec((bn, hid), lambda i: (i, 0)),
            pl.BlockSpec((2, bn), lambda i: (0, i)),
            pl.BlockSpec((1, hid), lambda i: (0, 0)),
            pl.BlockSpec((hid, ncls), lambda i: (0, 0)),
            pl.BlockSpec((1, ncls), lambda i: (0, 0)),
        ],
        out_specs=pl.BlockSpec((bn, ncls), lambda i: (i, 0)),
        out_shape=jax.ShapeDtypeStruct((n, ncls), jnp.float32),
    )(aggp, g, degp, b1.reshape(1, hid), W2, b2.reshape(1, ncls))


def kernel(x, edge_index, W1, b1, W2, b2):
    n, _ = x.shape
    hid = W1.shape[1]
    e = edge_index.shape[1]

    rt = 640
    n_pad = -(-(n + 1) // rt) * rt
    step = IB * NW * CHUNK
    e_pad = -(-e // step) * step
    cw = e_pad // (NW * CHUNK)
    bn = 512

    row = edge_index[0]
    col = edge_index[1]
    pad = e_pad - e
    if pad:
        ar = jnp.arange(pad, dtype=jnp.int32)
        row = jnp.concatenate([row, ar % n])
        col = jnp.concatenate([col, n + ar % (n_pad - n)])
    row3d = row.reshape(NW, cw, CHUNK)
    col3d = col.reshape(NW, cw, CHUNK)

    degp = _make_deg_kernel(n_pad, cw, rt)(col3d).reshape(NC, n_pad)
    g = _tc_scale(x, W1, degp, n_pad, bn)
    zrows = jnp.zeros((rt, hid), jnp.float32)
    aggp = _make_agg_kernel(n_pad, hid, cw, rt)(g, row3d, col3d, zrows)
    return _tc_head(aggp, g, degp, b1, W2, b2, n, n_pad, bn)


# speedup vs baseline: 1.1406x; 1.0176x over previous
import functools

import jax
import jax.numpy as jnp
from jax import lax
from jax.experimental import pallas as pl
from jax.experimental.pallas import tpu as pltpu
import jax.experimental.pallas.tpu_sc as plsc

NC = 2
NS = 16
NW = NC * NS
CHUNK = 128
IB = 16


def _sc_mesh():
    return plsc.VectorSubcoreMesh(
        core_axis_name="c", subcore_axis_name="s", num_cores=NC, num_subcores=NS
    )


def _make_deg_kernel(n_pad, cw, rt):
    @functools.partial(
        pl.kernel,
        mesh=_sc_mesh(),
        out_type=jax.ShapeDtypeStruct((NC, 1, n_pad), jnp.float32),
        scratch_types=[
            pltpu.VMEM((cw, CHUNK), jnp.int32),
            pltpu.VMEM((CHUNK,), jnp.float32),
            pltpu.VMEM((rt,), jnp.float32),
            pltpu.VMEM_SHARED((n_pad,), jnp.float32),
            pltpu.SemaphoreType.DMA,
        ],
    )
    def deg_k(col_hbm, out_hbm, col_v, ones_v, z_v, accd, ssem):
        cid = lax.axis_index("c")
        sid = lax.axis_index("s")
        wid = sid * NC + cid
        pltpu.sync_copy(col_hbm.at[wid], col_v)
        for k in range(CHUNK // 16):
            ones_v[pl.ds(16 * k, 16)] = jnp.full((16,), 1.0, jnp.float32)

        def zb(i, c):
            z_v[pl.ds(i * 16, 16)] = jnp.zeros((16,), jnp.float32)
            return c

        lax.fori_loop(0, rt // 16, zb, 0)
        pltpu.sync_copy(z_v, accd.at[pl.ds(sid * rt, rt)])
        plsc.subcore_barrier()

        def body(j, c):
            pltpu.async_copy(ones_v, accd.at[col_v.at[j]], ssem, add=True)
            return c

        lax.fori_loop(0, cw, body, 0)

        def drain(j, c):
            pltpu.make_async_copy(ones_v, accd.at[col_v.at[j]], ssem).wait()
            return c

        lax.fori_loop(0, cw, drain, 0)
        plsc.subcore_barrier()
        pltpu.sync_copy(
            accd.at[pl.ds(sid * rt, rt)], out_hbm.at[cid, 0, pl.ds(sid * rt, rt)]
        )

    return deg_k


def _make_agg_kernel(n_pad, hid, cw, rt):
    nblk = cw // IB
    assert cw % IB == 0 and IB % 8 == 0

    @functools.partial(
        pl.kernel,
        mesh=_sc_mesh(),
        out_type=jax.ShapeDtypeStruct((NC, n_pad, hid), jnp.float32),
        scratch_types=[
            pltpu.VMEM((2, IB, CHUNK), jnp.int32),
            pltpu.VMEM((2, IB, CHUNK), jnp.int32),
            pltpu.VMEM((2, CHUNK, hid), jnp.float32),
            pltpu.VMEM_SHARED((n_pad, hid), jnp.float32),
            pltpu.SemaphoreType.DMA,
            pltpu.SemaphoreType.DMA,
            pltpu.SemaphoreType.DMA,
        ],
    )
    def agg_k(g_hbm, row_hbm, col_hbm, zrows_hbm, out_hbm, row_v, col_v, buf,
              acc, gsem, ssem, isem):
        cid = lax.axis_index("c")
        sid = lax.axis_index("s")
        wid = sid * NC + cid
        pltpu.sync_copy(zrows_hbm, acc.at[pl.ds(sid * rt, rt)])
        pltpu.sync_copy(row_hbm.at[wid, pl.ds(0, IB)], row_v.at[0])
        pltpu.sync_copy(col_hbm.at[wid, pl.ds(0, IB)], col_v.at[0])
        pltpu.async_copy(g_hbm.at[row_v.at[0, 0]], buf.at[0], gsem)
        plsc.subcore_barrier()

        def half(k2, p):
            b = 2 * k2 + p

            @pl.when(b + 1 < nblk)
            def _():
                pltpu.async_copy(
                    row_hbm.at[wid, pl.ds((b + 1) * IB, IB)], row_v.at[1 - p],
                    isem)
                pltpu.async_copy(
                    col_hbm.at[wid, pl.ds((b + 1) * IB, IB)], col_v.at[1 - p],
                    isem)

            pltpu.async_copy(g_hbm.at[row_v.at[p, 1]], buf.at[1], gsem)

            def body(k, c2):
                j0 = 2 * k
                pltpu.make_async_copy(
                    g_hbm.at[row_v.at[p, j0]], buf.at[0], gsem
                ).wait()
                pltpu.async_copy(buf.at[0], acc.at[col_v.at[p, j0]], ssem,
                                 add=True)
                pltpu.make_async_copy(
                    g_hbm.at[row_v.at[p, j0 + 1]], buf.at[1], gsem
                ).wait()
                pltpu.async_copy(buf.at[1], acc.at[col_v.at[p, j0 + 1]], ssem,
                                 add=True)
                pltpu.make_async_copy(
                    buf.at[0], acc.at[col_v.at[p, j0]], ssem
                ).wait()

                @pl.when(j0 + 2 < IB)
                def _():
                    pltpu.async_copy(g_hbm.at[row_v.at[p, j0 + 2]], buf.at[0],
                                     gsem)

                pltpu.make_async_copy(
                    buf.at[1], acc.at[col_v.at[p, j0 + 1]], ssem
                ).wait()

                @pl.when(j0 + 3 < IB)
                def _():
                    pltpu.async_copy(g_hbm.at[row_v.at[p, j0 + 3]], buf.at[1],
                                     gsem)

                return c2

            lax.fori_loop(0, IB // 2, body, 0)

            @pl.when(b + 1 < nblk)
            def _():
                pltpu.make_async_copy(
                    row_hbm.at[wid, pl.ds((b + 1) * IB, IB)], row_v.at[1 - p],
                    isem).wait()
                pltpu.make_async_copy(
                    col_hbm.at[wid, pl.ds((b + 1) * IB, IB)], col_v.at[1 - p],
                    isem).wait()
                pltpu.async_copy(g_hbm.at[row_v.at[1 - p, 0]], buf.at[0], gsem)

        def outer(k2, c):
            half(k2, 0)
            half(k2, 1)
            return c

        lax.fori_loop(0, nblk // 2, outer, 0)
        if nblk % 2 == 1:
            half(nblk // 2, 0)
        plsc.subcore_barrier()
        pltpu.sync_copy(
            acc.at[pl.ds(sid * rt, rt)],
            out_hbm.at[cid, pl.ds(sid * rt, rt)],
        )

    return agg_k


def _tc_scale(x, W1, degp, n_pad, bn):
    f_in = x.shape[1]
    hid = W1.shape[1]

    def body(x_ref, w_ref, dp_ref, g_ref):
        h = jnp.dot(x_ref[...], w_ref[...], preferred_element_type=jnp.float32)
        deg = dp_ref[0, :] + dp_ref[1, :] + 1.0
        dis = jnp.where(deg > 0, lax.rsqrt(deg), 0.0)
        g_ref[...] = h * dis[:, None]

    return pl.pallas_call(
        body,
        grid=(n_pad // bn,),
        in_specs=[
            pl.BlockSpec((bn, f_in), lambda i: (i, 0)),
            pl.BlockSpec((f_in, hid), lambda i: (0, 0)),
            pl.BlockSpec((2, bn), lambda i: (0, i)),
        ],
        out_specs=pl.BlockSpec((bn, hid), lambda i: (i, 0)),
        out_shape=jax.ShapeDtypeStruct((n_pad, hid), jnp.float32),
    )(x, W1, degp)


def _tc_head(aggp, g, degp, b1, W2, b2, n, n_pad, bn):
    hid = g.shape[1]
    ncls = W2.shape[1]

    def body(a_ref, g_ref, dp_ref, b1_ref, w2_ref, b2_ref, o_ref):
        deg = dp_ref[0, :] + dp_ref[1, :] + 1.0
        dis = jnp.where(deg > 0, lax.rsqrt(deg), 0.0)
        s = a_ref[0] + a_ref[1] + g_ref[...]
        h = s * dis[:, None] + b1_ref[0, :][None, :]
        h = jnp.maximum(h, 0.0)
        logits = (
            jnp.dot(h, w2_ref[...], preferred_element_type=jnp.float32)
            + b2_ref[0, :][None, :]
        )
        m = jnp.max(logits, axis=1, keepdims=True)
        ex = logits - m
        lse = jnp.log(jnp.sum(jnp.exp(ex), axis=1, keepdims=True))
        o_ref[...] = ex - lse

    return pl.pallas_call(
        body,
        grid=(n_pad // bn,),
        in_specs=[
            pl.BlockSpec((2, bn, hid), lambda i: (0, i, 0)),
            pl.BlockSpec((bn, hid), lambda i: (i, 0)),
            pl.BlockSpec((2, bn), lambda i: (0, i)),
            pl.BlockSpec((1, hid), lambda i: (0, 0)),
            pl.BlockSpec((hid, ncls), lambda i: (0, 0)),
            pl.BlockSpec((1, ncls), lambda i: (0, 0)),
        ],
        out_specs=pl.BlockSpec((bn, ncls), lambda i: (i, 0)),
        out_shape=jax.ShapeDtypeStruct((n, ncls), jnp.float32),
    )(aggp, g, degp, b1.reshape(1, hid), W2, b2.reshape(1, ncls))


def kernel(x, edge_index, W1, b1, W2, b2):
    n, _ = x.shape
    hid = W1.shape[1]
    e = edge_index.shape[1]

    rt = 640
    n_pad = -(-(n + 1) // rt) * rt
    step = IB * NW * CHUNK
    e_pad = -(-e // step) * step
    cw = e_pad // (NW * CHUNK)
    bn = 512

    row = edge_index[0]
    col = edge_index[1]
    pad = e_pad - e
    if pad:
        ar = jnp.arange(pad, dtype=jnp.int32)
        row = jnp.concatenate([row, ar % n])
        col = jnp.concatenate([col, n + ar % (n_pad - n)])
    row3d = row.reshape(NW, cw, CHUNK)
    col3d = col.reshape(NW, cw, CHUNK)

    degp = _make_deg_kernel(n_pad, cw, rt)(col3d).reshape(NC, n_pad)
    g = _tc_scale(x, W1, degp, n_pad, bn)
    zrows = jnp.zeros((rt, hid), jnp.float32)
    aggp = _make_agg_kernel(n_pad, hid, cw, rt)(g, row3d, col3d, zrows)
    return _tc_head(aggp, g, degp, b1, W2, b2, n, n_pad, bn)
